# baseline (speedup 1.0000x reference)
import functools

import jax
import jax.numpy as jnp
from jax import lax
from jax.experimental import pallas as pl
from jax.experimental.pallas import tpu as pltpu
from jax.experimental.pallas import tpu_sc as plsc

VOCAB = 1000
B, T = 1024, 200
N = B * T
NC, NS = 2, 16
NW = NC * NS
PER_W = N // NW
CHUNK = 40
NCHUNK = PER_W // CHUNK

_mesh = plsc.VectorSubcoreMesh(core_axis_name="c", subcore_axis_name="s")


@functools.partial(
    pl.kernel,
    out_type=jax.ShapeDtypeStruct((N, VOCAB), jnp.float32),
    mesh=_mesh,
    compiler_params=pltpu.CompilerParams(use_tc_tiling_on_sc=False),
    scratch_types=[
        pltpu.VMEM((PER_W,), jnp.int32),
        pltpu.VMEM((CHUNK, VOCAB), jnp.float32),
        pltpu.VMEM((CHUNK, VOCAB), jnp.float32),
        pltpu.VMEM((CHUNK, VOCAB), jnp.float32),
        pltpu.SemaphoreType.DMA,
        pltpu.SemaphoreType.DMA,
        pltpu.SemaphoreType.DMA,
        pltpu.SemaphoreType.DMA,
        pltpu.SemaphoreType.DMA,
        pltpu.SemaphoreType.DMA,
    ],
)
def _gather_rows(idx_hbm, table_hbm, out_hbm, idx_v, row0, row1, row2,
                 gsem0, gsem1, gsem2, osem0, osem1, osem2):
    wid = lax.axis_index("s") * NC + lax.axis_index("c")
    base = wid * PER_W
    pltpu.sync_copy(idx_hbm.at[pl.ds(base, PER_W)], idx_v)

    bufs = (row0, row1, row2)
    gsems = (gsem0, gsem1, gsem2)
    osems = (osem0, osem1, osem2)

    def gather(i, b):
        return pltpu.make_async_copy(
            table_hbm.at[idx_v.at[pl.ds(i * CHUNK, CHUNK)]],
            bufs[b], gsems[b])

    def outcopy(i, b):
        return pltpu.make_async_copy(
            bufs[b], out_hbm.at[pl.ds(base + i * CHUNK, CHUNK)], osems[b])

    for b in range(3):
        gather(b, b).start()
    gather(0, 0).wait()
    outcopy(0, 0).start()

    def body(p, carry):
        for b in range(3):
            i = 3 * p + b + 1
            bg = (b + 1) % 3

            @pl.when(i + 2 < NCHUNK)
            def _():
                outcopy(i - 1, b).wait()
                gather(i + 2, b).start()

            @pl.when(i < NCHUNK)
            def _():
                gather(i, bg).wait()
                outcopy(i, bg).start()
        return carry

    lax.fori_loop(0, (NCHUNK + 1) // 3, body, 0)

    for j in range(NCHUNK - 3, NCHUNK):
        outcopy(j, j % 3).wait()


def kernel(idx, token_embedding_table):
    flat = _gather_rows(idx.reshape(N).astype(jnp.int32),
                        token_embedding_table)
    return flat.reshape(B, T, VOCAB)


# speedup vs baseline: 1.7373x; 1.7373x over previous
import functools

import jax
import jax.numpy as jnp
from jax import lax
from jax.experimental import pallas as pl
from jax.experimental.pallas import tpu as pltpu
from jax.experimental.pallas import tpu_sc as plsc

VOCAB = 1000
VPAD = 1024
NTILE = VPAD // 128
B, T = 1024, 200
N = B * T
NC, NS = 2, 16
NW = NC * NS
PER_W = N // NW
CHUNK = 32
NCHUNK = PER_W // CHUNK
NVR = CHUNK // 16

_mesh = plsc.VectorSubcoreMesh(core_axis_name="c", subcore_axis_name="s")


@functools.partial(
    pl.kernel,
    out_type=jax.ShapeDtypeStruct((N, VOCAB), jnp.float32),
    mesh=_mesh,
    compiler_params=pltpu.CompilerParams(needs_layout_passes=False),
    scratch_types=[
        pltpu.VMEM((PER_W,), jnp.int32),
        pltpu.VMEM((3, NTILE, CHUNK), jnp.int32),
        pltpu.VMEM((CHUNK, VOCAB), jnp.float32),
        pltpu.VMEM((CHUNK, VOCAB), jnp.float32),
        pltpu.VMEM((CHUNK, VOCAB), jnp.float32),
        pltpu.VMEM((3, CHUNK, 128), jnp.float32),
        pltpu.SemaphoreType.DMA,
        pltpu.SemaphoreType.DMA,
        pltpu.SemaphoreType.DMA,
        pltpu.SemaphoreType.DMA,
        pltpu.SemaphoreType.DMA,
        pltpu.SemaphoreType.DMA,
    ],
)
def _gather_rows(idx_hbm, table_hbm, out_hbm, idx_v, ibuf, ob0, ob1, ob2,
                 tbuf, gsem0, gsem1, gsem2, osem0, osem1, osem2):
    wid = lax.axis_index("s") * NC + lax.axis_index("c")
    base = wid * PER_W
    pltpu.sync_copy(idx_hbm.at[pl.ds(base, PER_W)], idx_v)

    obufs = (ob0, ob1, ob2)
    gsems = (gsem0, gsem1, gsem2)
    osems = (osem0, osem1, osem2)

    def gathers(i, b):
        cps = []
        for tc in range(NTILE - 1):
            cps.append(pltpu.make_async_copy(
                table_hbm.at[ibuf.at[b, tc]],
                obufs[b].at[:, pl.ds(128 * tc, 128)], gsems[b]))
        cps.append(pltpu.make_async_copy(
            table_hbm.at[ibuf.at[b, NTILE - 1]], tbuf.at[b], gsems[b]))
        return cps

    def outcopy(i, b):
        return pltpu.make_async_copy(
            obufs[b], out_hbm.at[pl.ds(base + i * CHUNK, CHUNK)], osems[b])

    def issue(i, b):
        vs = [idx_v[pl.ds(i * CHUNK + 16 * k, 16)] for k in range(NVR)]
        for tc in range(NTILE):
            for k in range(NVR):
                ibuf[b, tc, pl.ds(16 * k, 16)] = vs[k] + 8 * tc
        for cp in gathers(i, b):
            cp.start()

    def drain(i, b):
        for cp in gathers(i, b):
            cp.wait()

        lanes = lax.iota(jnp.int32, 16)
        tail_cols = 992 + lanes
        tail_mask = lanes < 8

        def row_move(r, carry):
            for k in range(6):
                obufs[b][r, pl.ds(896 + 16 * k, 16)] = tbuf[b, r, pl.ds(16 * k, 16)]
            x = tbuf[b, r, pl.ds(96, 16)]
            rvec = jnp.full((16,), r, jnp.int32)
            plsc.store_scatter(obufs[b], [rvec, tail_cols], x, mask=tail_mask)
            return carry

        lax.fori_loop(0, CHUNK, row_move, 0)
        outcopy(i, b).start()

    for b in range(3):
        issue(b, b)
    drain(0, 0)

    def body(p, carry):
        for b in range(3):
            i = 3 * p + b + 1
            bg = (b + 1) % 3

            @pl.when(i + 2 < NCHUNK)
            def _():
                outcopy(i - 1, b).wait()
                issue(i + 2, b)

            @pl.when(i < NCHUNK)
            def _():
                drain(i, bg)
        return carry

    lax.fori_loop(0, (NCHUNK + 1) // 3, body, 0)

    for j in range(NCHUNK - 3, NCHUNK):
        outcopy(j, j % 3).wait()


def kernel(idx, token_embedding_table):
    pad = jnp.pad(token_embedding_table, ((0, 0), (0, VPAD - VOCAB)))
    table_r = pad.reshape(125, 8, NTILE, 128).transpose(0, 2, 1, 3).reshape(8000, 128)
    flat_idx = idx.reshape(N).astype(jnp.int32)
    base_idx = (flat_idx // 8) * 64 + flat_idx % 8
    flat = _gather_rows(base_idx, table_r)
    return flat.reshape(B, T, VOCAB)
